# baseline (speedup 1.0000x reference)
import functools

import jax
import jax.numpy as jnp
from jax import lax
from jax.experimental import pallas as pl
from jax.experimental.pallas import tpu as pltpu
from jax.experimental.pallas import tpu_sc as plsc

N = 10000
E = 320000
D = 128
NPAD = 10240
NC = 2
NS = 16
CH = 128
KC = 160
E2 = NS * KC * CH
RPT = NPAD // NS

_MESH = plsc.VectorSubcoreMesh(
    core_axis_name="c", subcore_axis_name="s", num_cores=NC, num_subcores=NS)


@functools.partial(
    pl.kernel,
    out_type=jax.ShapeDtypeStruct((NC, NPAD), jnp.float32),
    mesh=_MESH,
    scratch_types=[
        pltpu.VMEM((KC, CH), jnp.int32),
        pltpu.VMEM((CH,), jnp.float32),
        pltpu.VMEM((CH,), jnp.float32),
        pltpu.VMEM_SHARED((NPAD,), jnp.float32),
    ],
)
def _deg_kernel(idx_hbm, out_hbm, idx_v, ones_v, zeros_v, acc_sh):
    c = lax.axis_index("c")
    s = lax.axis_index("s")
    for j in range(CH // 16):
        ones_v[pl.ds(j * 16, 16)] = jnp.ones((16,), jnp.float32)
        zeros_v[pl.ds(j * 16, 16)] = jnp.zeros((16,), jnp.float32)
    for t in range(RPT // CH):
        pltpu.sync_copy(zeros_v, acc_sh.at[pl.ds(s * RPT + t * CH, CH)])
    plsc.subcore_barrier()
    pltpu.sync_copy(idx_hbm.at[c, pl.ds(s * KC, KC)], idx_v)

    def body(k, carry):
        pltpu.sync_copy(ones_v, acc_sh.at[idx_v.at[k]], add=True)
        return carry

    lax.fori_loop(0, KC, body, 0)
    plsc.subcore_barrier()
    for t in range(RPT // CH):
        r0 = s * RPT + t * CH
        pltpu.sync_copy(acc_sh.at[pl.ds(r0, CH)], out_hbm.at[c, pl.ds(r0, CH)])


SCH = 16
NSC = KC // SCH


HROWS = CH // 2


@functools.partial(
    pl.kernel,
    out_type=jax.ShapeDtypeStruct((2 * NC, NPAD, D), jnp.float32),
    mesh=_MESH,
    scratch_types=[
        pltpu.VMEM((SCH, CH), jnp.int32),
        pltpu.VMEM((2 * SCH, HROWS), jnp.int32),
        pltpu.VMEM((SCH * CH + 16,), jnp.float32),
        pltpu.VMEM((CH, D // 2), jnp.int32),
        pltpu.VMEM((CH, D // 2), jnp.int32),
        pltpu.VMEM((HROWS, D), jnp.float32),
        pltpu.VMEM((HROWS, D), jnp.float32),
        pltpu.VMEM_SHARED((NPAD, D), jnp.float32),
        pltpu.SemaphoreType.DMA,
        pltpu.SemaphoreType.DMA,
        pltpu.SemaphoreType.DMA,
        pltpu.SemaphoreType.DMA,
    ],
    compiler_params=pltpu.CompilerParams(use_tc_tiling_on_sc=False),
)
def _agg_kernel(gidx_hbm, sidx_hbm, ew_hbm, tbl_hbm, out_hbm,
                gix_v, six_v, ew_v, pbuf0, pbuf1, st0, st1, acc_sh,
                gsem0, gsem1, ssem0, ssem1):
    c = lax.axis_index("c")
    s = lax.axis_index("s")
    pbufs = (pbuf0, pbuf1)
    stg = (st0, st1)
    gsems = (gsem0, gsem1)
    ssems = (ssem0, ssem1)
    hmask = jnp.full((16,), -65536, jnp.int32)

    def zero_st(i, carry):
        for j in range(D // 16):
            st0[i, pl.ds(j * 16, 16)] = jnp.zeros((16,), jnp.float32)
        return carry

    for r in range(2):
        lax.fori_loop(0, HROWS, zero_st, 0)
        for t in range(RPT // HROWS):
            pltpu.sync_copy(st0,
                            acc_sh.at[pl.ds(s * RPT + t * HROWS, HROWS)])
        plsc.subcore_barrier()

        plane = 2 * r + c

        def unpack_half(pbuf, h, st, k):
            def grp(g, carry):
                for l in range(4):
                    i = g * 4 + l
                    if r == 1:
                        w = ew_v[pl.ds(k * CH + h * HROWS + i, 16)][0]
                    for j in range(D // 32):
                        v = pbuf[h * HROWS + i, pl.ds(j * 16, 16)]
                        lof = lax.bitcast_convert_type(v << 16, jnp.float32)
                        hif = lax.bitcast_convert_type(v & hmask, jnp.float32)
                        if r == 1:
                            lof = lof * w
                            hif = hif * w
                        st[i, pl.ds(j * 16, 16)] = lof
                        st[i, pl.ds(64 + j * 16, 16)] = hif
                return carry
            lax.fori_loop(0, HROWS // 4, grp, 0)

        def superchunk(q, carry):
            @pl.when(q > 0)
            def _():
                pltpu.make_async_copy(
                    st0, acc_sh.at[six_v.at[0]], ssems[0]).wait()
                pltpu.make_async_copy(
                    st1, acc_sh.at[six_v.at[1]], ssems[1]).wait()

            base_row = s * KC + q * SCH
            pltpu.sync_copy(gidx_hbm.at[plane, pl.ds(base_row, SCH)], gix_v)
            pltpu.sync_copy(sidx_hbm.at[r, pl.ds(2 * base_row, 2 * SCH)],
                            six_v)
            if r == 1:
                pltpu.sync_copy(ew_hbm.at[pl.ds(base_row * CH, SCH * CH)],
                                ew_v.at[pl.ds(0, SCH * CH)])
            pltpu.async_copy(tbl_hbm.at[gix_v.at[0]], pbuf0, gsem0)

            def pair(p, carry2):
                k0 = 2 * p
                pltpu.make_async_copy(
                    tbl_hbm.at[gix_v.at[k0]], pbuf0, gsem0).wait()
                pltpu.async_copy(tbl_hbm.at[gix_v.at[k0 + 1]], pbuf1, gsem1)
                for h in range(2):
                    @pl.when(k0 > 0)
                    def _():
                        pltpu.make_async_copy(
                            stg[h], acc_sh.at[six_v.at[h]], ssems[h]).wait()
                    unpack_half(pbuf0, h, stg[h], k0)
                    pltpu.async_copy(stg[h],
                                     acc_sh.at[six_v.at[2 * k0 + h]],
                                     ssems[h], add=True)
                pltpu.make_async_copy(
                    tbl_hbm.at[gix_v.at[k0 + 1]], pbuf1, gsem1).wait()

                @pl.when(k0 + 2 < SCH)
                def _():
                    pltpu.async_copy(tbl_hbm.at[gix_v.at[k0 + 2]],
                                     pbuf0, gsem0)
                for h in range(2):
                    pltpu.make_async_copy(
                        stg[h], acc_sh.at[six_v.at[h]], ssems[h]).wait()
                    unpack_half(pbuf1, h, stg[h], k0 + 1)
                    pltpu.async_copy(stg[h],
                                     acc_sh.at[six_v.at[2 * k0 + 2 + h]],
                                     ssems[h], add=True)
                return carry2

            lax.fori_loop(0, SCH // 2, pair, 0)
            return carry

        lax.fori_loop(0, NSC, superchunk, 0)
        pltpu.make_async_copy(st0, acc_sh.at[six_v.at[0]], ssems[0]).wait()
        pltpu.make_async_copy(st1, acc_sh.at[six_v.at[1]], ssems[1]).wait()

        plsc.subcore_barrier()
        for t in range(RPT // CH):
            r0 = s * RPT + t * CH
            pltpu.sync_copy(acc_sh.at[pl.ds(r0, CH)],
                            out_hbm.at[plane, pl.ds(r0, CH)])


def _pack_rows(x):
    xb = lax.bitcast_convert_type(x, jnp.uint32)
    t = (xb + jnp.uint32(0x7FFF) + ((xb >> 16) & jnp.uint32(1))) >> 16
    pk = t[:, :64] | (t[:, 64:] << 16)
    return lax.bitcast_convert_type(pk, jnp.int32)


def _proj_body(fa, fb, sa, sb, wa, wb, ba, bb, deg, out):
    pa = jnp.dot(fa[...], wa[...], preferred_element_type=jnp.float32) + ba[...]
    pb = jnp.dot(fb[...], wb[...], preferred_element_type=jnp.float32) + bb[...]
    proj = jnp.concatenate([pa, pb], axis=1)
    qa = jnp.dot(sa[...], wa[...], preferred_element_type=jnp.float32) + ba[...]
    qb = jnp.dot(sb[...], wb[...], preferred_element_type=jnp.float32) + bb[...]
    shuf = jnp.concatenate([qa, qb], axis=1)
    dg = deg[...]
    ns = jnp.where(dg > 0, lax.rsqrt(jnp.maximum(dg, 1.0)), 0.0)
    out[0] = _pack_rows(proj * ns)
    out[1] = _pack_rows(shuf * ns)
    out[2] = _pack_rows(proj)
    out[3] = _pack_rows(shuf)


_BLK = 1000


def _proj_call(fa, fb, sa, sb, wa, wb, ba, bb, deg_col):
    grid = N // _BLK
    row_spec = pl.BlockSpec((_BLK, 128), lambda i: (i, 0))
    full64 = pl.BlockSpec((128, 64), lambda i: (0, 0))
    bias = pl.BlockSpec((1, 64), lambda i: (0, 0))
    return pl.pallas_call(
        _proj_body,
        grid=(grid,),
        in_specs=[row_spec, row_spec, row_spec, row_spec,
                  full64, full64, bias, bias,
                  pl.BlockSpec((_BLK, 1), lambda i: (i, 0))],
        out_specs=pl.BlockSpec((4, _BLK, 64), lambda i: (0, i, 0)),
        out_shape=jax.ShapeDtypeStruct((4, N, 64), jnp.int32),
    )(fa, fb, sa, sb, wa, wb, ba, bb, deg_col)


def _encode(a0, a2, deg, w1, b1, w2, b2, a1r, a2r):
    nd = jnp.where(deg > 0, lax.rsqrt(jnp.maximum(deg, 1.0)), 0.0)
    z1 = jnp.dot(a0, w1, preferred_element_type=jnp.float32) * nd + b1
    z2 = jnp.dot(a2, w2, preferred_element_type=jnp.float32) + b2
    h1 = jnp.where(z1 > 0, z1, a1r * z1)
    h2 = jnp.where(z2 > 0, z2, a2r * z2)
    return h1, h2


def _sums_body(a0, a2, deg, w1, b1, w2, b2, a1r, a2r, wbil, out, acc):
    i = pl.program_id(0)

    @pl.when(i == 0)
    def _():
        acc[...] = jnp.zeros_like(acc)

    h1, h2 = _encode(a0[...], a2[...], deg[...], w1[...], b1[...],
                     w2[...], b2[...], a1r[...], a2r[...])
    acc[0:1, :] += jnp.sum(h1, axis=0, keepdims=True)
    acc[1:2, :] += jnp.sum(h2, axis=0, keepdims=True)

    @pl.when(i == N // _BLK - 1)
    def _():
        c = jax.nn.sigmoid(acc[...] * (1.0 / N))
        eye = jnp.eye(128, dtype=jnp.float32)
        dnt = (((1,), (1,)), ((), ()))
        cc = lax.dot_general(eye, c, dnt,
                             preferred_element_type=jnp.float32)
        out[...] = jnp.dot(wbil[...], cc, preferred_element_type=jnp.float32)


def _sums_call(agg, deg_col, w1, b1, w2, b2, a1r, a2r, wbil):
    grid = N // _BLK
    blk = pl.BlockSpec((_BLK, 128), lambda i: (i, 0))
    full = pl.BlockSpec((128, 128), lambda i: (0, 0))
    row = pl.BlockSpec((1, 128), lambda i: (0, 0))
    one = pl.BlockSpec((1, 1), lambda i: (0, 0))
    return pl.pallas_call(
        _sums_body,
        grid=(grid,),
        in_specs=[blk, blk, pl.BlockSpec((_BLK, 1), lambda i: (i, 0)),
                  full, row, full, row, one, one, full],
        out_specs=pl.BlockSpec((128, 2), lambda i: (0, 0)),
        out_shape=jax.ShapeDtypeStruct((128, 2), jnp.float32),
        scratch_shapes=[pltpu.VMEM((2, 128), jnp.float32)],
    )(agg[0], agg[2], deg_col, w1, b1, w2, b2, a1r, a2r, wbil)


def _scores_body(agg, deg, w1, b1, w2, b2, a1r, a2r, vb, bbr, out):
    h1, h2 = _encode(agg[0], agg[2], deg[...], w1[...], b1[...],
                     w2[...], b2[...], a1r[...], a2r[...])
    h3, h4 = _encode(agg[1], agg[3], deg[...], w1[...], b1[...],
                     w2[...], b2[...], a1r[...], a2r[...])
    v = vb[...]
    p1 = jnp.dot(h1, v, preferred_element_type=jnp.float32)
    p2 = jnp.dot(h2, v, preferred_element_type=jnp.float32)
    p3 = jnp.dot(h3, v, preferred_element_type=jnp.float32)
    p4 = jnp.dot(h4, v, preferred_element_type=jnp.float32)
    out[...] = jnp.concatenate(
        [p2[:, 0:1], p1[:, 1:2], p4[:, 0:1], p3[:, 1:2]], axis=1) + bbr[...]


def _scores_call(agg, deg_col, w1, b1, w2, b2, a1r, a2r, vb, bb4):
    grid = N // _BLK
    blk = pl.BlockSpec((4, _BLK, 128), lambda i: (0, i, 0))
    full = pl.BlockSpec((128, 128), lambda i: (0, 0))
    row = pl.BlockSpec((1, 128), lambda i: (0, 0))
    one = pl.BlockSpec((1, 1), lambda i: (0, 0))
    return pl.pallas_call(
        _scores_body,
        grid=(grid,),
        in_specs=[blk, pl.BlockSpec((_BLK, 1), lambda i: (i, 0)),
                  full, row, full, row, one, one,
                  pl.BlockSpec((128, 2), lambda i: (0, 0)),
                  pl.BlockSpec((1, 4), lambda i: (0, 0))],
        out_specs=pl.BlockSpec((_BLK, 4), lambda i: (i, 0)),
        out_shape=jax.ShapeDtypeStruct((N, 4), jnp.float32),
    )(agg, deg_col, w1, b1, w2, b2, a1r, a2r, vb, bb4)


def kernel(edge_index, diff_edge_index, feat_a, feat_b, shuf_feat_a,
           shuf_feat_b, edge_weight, Wa, ba, Wb, bb, W1, b1, alpha1,
           W2, b2, alpha2, Wbil, bbil):
    src1, dst1 = edge_index[0], edge_index[1]
    src2, dst2 = diff_edge_index[0], diff_edge_index[1]
    pad = E2 - E
    pad_src = jnp.arange(pad, dtype=jnp.int32) % N
    pad_sink = jnp.full((pad,), NPAD - 1, jnp.int32)

    deg_idx = jnp.stack([
        jnp.concatenate([src1, pad_sink]),
        jnp.concatenate([dst1, pad_sink]),
    ]).reshape(NC, NS * KC, CH)
    deg = _deg_kernel(deg_idx)

    src1p = jnp.concatenate([src1, pad_src])
    src2p = jnp.concatenate([src2, pad_src])
    dst1p = jnp.concatenate([dst1, pad_sink])
    dst2p = jnp.concatenate([dst2, pad_sink])
    gidx = jnp.stack([src1p, src1p + N, src2p + 2 * N, src2p + 3 * N]
                     ).reshape(4, NS * KC, CH)
    sidx = jnp.stack([dst1p, dst2p]).reshape(2, NS * KC * 2, HROWS)
    ewp = jnp.concatenate([edge_weight, jnp.zeros((pad,), jnp.float32)])

    tbl = _proj_call(feat_a, feat_b, shuf_feat_a, shuf_feat_b,
                     Wa, Wb, ba.reshape(1, 64), bb.reshape(1, 64),
                     deg[0, :N].reshape(N, 1)).reshape(4 * N, 64)

    agg = _agg_kernel(gidx, sidx, ewp, tbl)

    deg_in = deg[1].reshape(NPAD, 1)
    b1r = b1.reshape(1, 128)
    b2r = b2.reshape(1, 128)
    a1r = alpha1.reshape(1, 1)
    a2r = alpha2.reshape(1, 1)
    vb = _sums_call(agg, deg_in, W1, b1r, W2, b2r, a1r, a2r, Wbil)
    bb4 = jnp.broadcast_to(bbil.reshape(1, 1), (1, 4))
    scores = _scores_call(agg, deg_in, W1, b1r, W2, b2r, a1r, a2r, vb, bb4)
    return scores.T.reshape(-1)


# speedup vs baseline: 1.6757x; 1.6757x over previous
import functools

import jax
import jax.numpy as jnp
from jax import lax
from jax.experimental import pallas as pl
from jax.experimental.pallas import tpu as pltpu
from jax.experimental.pallas import tpu_sc as plsc

N = 10000
E = 320000
D = 128
NPAD = 10240
NC = 2
NS = 16
CH = 128
KC = 160
E2 = NS * KC * CH
RPT = NPAD // NS

_MESH = plsc.VectorSubcoreMesh(
    core_axis_name="c", subcore_axis_name="s", num_cores=NC, num_subcores=NS)


@functools.partial(
    pl.kernel,
    out_type=jax.ShapeDtypeStruct((NC, NPAD), jnp.float32),
    mesh=_MESH,
    scratch_types=[
        pltpu.VMEM((KC, CH), jnp.int32),
        pltpu.VMEM((CH,), jnp.float32),
        pltpu.VMEM((CH,), jnp.float32),
        pltpu.VMEM_SHARED((NPAD,), jnp.float32),
    ],
)
def _deg_kernel(idx_hbm, out_hbm, idx_v, ones_v, zeros_v, acc_sh):
    c = lax.axis_index("c")
    s = lax.axis_index("s")
    for j in range(CH // 16):
        ones_v[pl.ds(j * 16, 16)] = jnp.ones((16,), jnp.float32)
        zeros_v[pl.ds(j * 16, 16)] = jnp.zeros((16,), jnp.float32)
    for t in range(RPT // CH):
        pltpu.sync_copy(zeros_v, acc_sh.at[pl.ds(s * RPT + t * CH, CH)])
    plsc.subcore_barrier()
    pltpu.sync_copy(idx_hbm.at[c, pl.ds(s * KC, KC)], idx_v)

    def body(k, carry):
        pltpu.sync_copy(ones_v, acc_sh.at[idx_v.at[k]], add=True)
        return carry

    lax.fori_loop(0, KC, body, 0)
    plsc.subcore_barrier()
    for t in range(RPT // CH):
        r0 = s * RPT + t * CH
        pltpu.sync_copy(acc_sh.at[pl.ds(r0, CH)], out_hbm.at[c, pl.ds(r0, CH)])


SCH = 40
NSC = KC // SCH


@functools.partial(
    pl.kernel,
    out_type=jax.ShapeDtypeStruct((2 * NC, NPAD, D), jnp.float32),
    mesh=_MESH,
    scratch_types=[
        pltpu.VMEM((SCH, CH), jnp.int32),
        pltpu.VMEM((SCH, CH), jnp.int32),
        pltpu.VMEM((SCH * CH,), jnp.float32),
        pltpu.VMEM((CH, D), jnp.float32),
        pltpu.VMEM((CH, D), jnp.float32),
        pltpu.VMEM_SHARED((NPAD, D), jnp.float32),
        pltpu.SemaphoreType.DMA,
        pltpu.SemaphoreType.DMA,
        pltpu.SemaphoreType.DMA,
        pltpu.SemaphoreType.DMA,
    ],
)
def _agg_kernel(gidx_hbm, sidx_hbm, ew_hbm, tbl_hbm, out_hbm,
                gix_v, six_v, ew_v, buf0, buf1, acc_sh,
                sem0, sem1, ssem0, ssem1):
    c = lax.axis_index("c")
    s = lax.axis_index("s")
    bufs = (buf0, buf1)
    gsems = (sem0, sem1)
    ssems = (ssem0, ssem1)

    def zero_buf(i, carry):
        for j in range(D // 16):
            buf0[i, pl.ds(j * 16, 16)] = jnp.zeros((16,), jnp.float32)
        return carry

    for r in range(2):
        lax.fori_loop(0, CH, zero_buf, 0)
        for t in range(RPT // CH):
            pltpu.sync_copy(buf0, acc_sh.at[pl.ds(s * RPT + t * CH, CH)])
        plsc.subcore_barrier()

        plane = 2 * r + c

        def scale_rows(k, buf):
            def row16(ii, carry):
                wv = ew_v[pl.ds(k * CH + ii * 16, 16)]
                for l in range(16):
                    w = wv[l]
                    i = ii * 16 + l
                    for j in range(D // 16):
                        buf[i, pl.ds(j * 16, 16)] = (
                            buf[i, pl.ds(j * 16, 16)] * w)
                return carry
            lax.fori_loop(0, CH // 16, row16, 0)

        def superchunk(q, carry):
            base_row = s * KC + q * SCH
            pltpu.sync_copy(gidx_hbm.at[plane, pl.ds(base_row, SCH)], gix_v)
            pltpu.sync_copy(sidx_hbm.at[r, pl.ds(base_row, SCH)], six_v)
            if r == 1:
                pltpu.sync_copy(ew_hbm.at[pl.ds(base_row * CH, SCH * CH)],
                                ew_v)
            pltpu.async_copy(tbl_hbm.at[gix_v.at[0]], buf0, sem0)

            def pair(p, carry2):
                k0 = 2 * p
                pltpu.make_async_copy(
                    tbl_hbm.at[gix_v.at[k0]], buf0, gsems[0]).wait()

                @pl.when(k0 > 0)
                def _():
                    pltpu.make_async_copy(
                        buf1, acc_sh.at[six_v.at[0]], ssems[1]).wait()
                pltpu.async_copy(tbl_hbm.at[gix_v.at[k0 + 1]],
                                 buf1, gsems[1])
                if r == 1:
                    scale_rows(k0, buf0)
                pltpu.async_copy(buf0, acc_sh.at[six_v.at[k0]],
                                 ssems[0], add=True)
                pltpu.make_async_copy(
                    tbl_hbm.at[gix_v.at[k0 + 1]], buf1, gsems[1]).wait()
                pltpu.make_async_copy(
                    buf0, acc_sh.at[six_v.at[0]], ssems[0]).wait()

                @pl.when(k0 + 2 < SCH)
                def _():
                    pltpu.async_copy(tbl_hbm.at[gix_v.at[k0 + 2]],
                                     buf0, gsems[0])
                if r == 1:
                    scale_rows(k0 + 1, buf1)
                pltpu.async_copy(buf1, acc_sh.at[six_v.at[k0 + 1]],
                                 ssems[1], add=True)
                return carry2

            lax.fori_loop(0, SCH // 2, pair, 0)
            pltpu.make_async_copy(
                buf1, acc_sh.at[six_v.at[0]], ssems[1]).wait()
            return carry

        lax.fori_loop(0, NSC, superchunk, 0)

        plsc.subcore_barrier()
        for t in range(RPT // CH):
            r0 = s * RPT + t * CH
            pltpu.sync_copy(acc_sh.at[pl.ds(r0, CH)],
                            out_hbm.at[plane, pl.ds(r0, CH)])


def _proj_body(fa, fb, sa, sb, wa, wb, ba, bb, deg, out):
    pa = jnp.dot(fa[...], wa[...], preferred_element_type=jnp.float32) + ba[...]
    pb = jnp.dot(fb[...], wb[...], preferred_element_type=jnp.float32) + bb[...]
    proj = jnp.concatenate([pa, pb], axis=1)
    qa = jnp.dot(sa[...], wa[...], preferred_element_type=jnp.float32) + ba[...]
    qb = jnp.dot(sb[...], wb[...], preferred_element_type=jnp.float32) + bb[...]
    shuf = jnp.concatenate([qa, qb], axis=1)
    dg = deg[...]
    ns = jnp.where(dg > 0, lax.rsqrt(jnp.maximum(dg, 1.0)), 0.0)
    out[0] = proj * ns
    out[1] = shuf * ns
    out[2] = proj
    out[3] = shuf


_BLK = 1000


def _proj_call(fa, fb, sa, sb, wa, wb, ba, bb, deg_col):
    grid = N // _BLK
    row_spec = pl.BlockSpec((_BLK, 128), lambda i: (i, 0))
    full64 = pl.BlockSpec((128, 64), lambda i: (0, 0))
    bias = pl.BlockSpec((1, 64), lambda i: (0, 0))
    return pl.pallas_call(
        _proj_body,
        grid=(grid,),
        in_specs=[row_spec, row_spec, row_spec, row_spec,
                  full64, full64, bias, bias,
                  pl.BlockSpec((_BLK, 1), lambda i: (i, 0))],
        out_specs=pl.BlockSpec((4, _BLK, 128), lambda i: (0, i, 0)),
        out_shape=jax.ShapeDtypeStruct((4, N, 128), jnp.float32),
    )(fa, fb, sa, sb, wa, wb, ba, bb, deg_col)


def _encode(a0, a2, deg, w1, b1, w2, b2, a1r, a2r):
    nd = jnp.where(deg > 0, lax.rsqrt(jnp.maximum(deg, 1.0)), 0.0)
    z1 = jnp.dot(a0, w1, preferred_element_type=jnp.float32) * nd + b1
    z2 = jnp.dot(a2, w2, preferred_element_type=jnp.float32) + b2
    h1 = jnp.where(z1 > 0, z1, a1r * z1)
    h2 = jnp.where(z2 > 0, z2, a2r * z2)
    return h1, h2


def _sums_body(a0, a2, deg, w1, b1, w2, b2, a1r, a2r, wbil, out, acc):
    i = pl.program_id(0)

    @pl.when(i == 0)
    def _():
        acc[...] = jnp.zeros_like(acc)

    h1, h2 = _encode(a0[...], a2[...], deg[...], w1[...], b1[...],
                     w2[...], b2[...], a1r[...], a2r[...])
    acc[0:1, :] += jnp.sum(h1, axis=0, keepdims=True)
    acc[1:2, :] += jnp.sum(h2, axis=0, keepdims=True)

    @pl.when(i == N // _BLK - 1)
    def _():
        c = jax.nn.sigmoid(acc[...] * (1.0 / N))
        eye = jnp.eye(128, dtype=jnp.float32)
        dnt = (((1,), (1,)), ((), ()))
        cc = lax.dot_general(eye, c, dnt,
                             preferred_element_type=jnp.float32)
        out[...] = jnp.dot(wbil[...], cc, preferred_element_type=jnp.float32)


def _sums_call(agg, deg_col, w1, b1, w2, b2, a1r, a2r, wbil):
    grid = N // _BLK
    blk = pl.BlockSpec((_BLK, 128), lambda i: (i, 0))
    full = pl.BlockSpec((128, 128), lambda i: (0, 0))
    row = pl.BlockSpec((1, 128), lambda i: (0, 0))
    one = pl.BlockSpec((1, 1), lambda i: (0, 0))
    return pl.pallas_call(
        _sums_body,
        grid=(grid,),
        in_specs=[blk, blk, pl.BlockSpec((_BLK, 1), lambda i: (i, 0)),
                  full, row, full, row, one, one, full],
        out_specs=pl.BlockSpec((128, 2), lambda i: (0, 0)),
        out_shape=jax.ShapeDtypeStruct((128, 2), jnp.float32),
        scratch_shapes=[pltpu.VMEM((2, 128), jnp.float32)],
    )(agg[0], agg[2], deg_col, w1, b1, w2, b2, a1r, a2r, wbil)


def _scores_body(agg, deg, w1, b1, w2, b2, a1r, a2r, vb, bbr, out):
    h1, h2 = _encode(agg[0], agg[2], deg[...], w1[...], b1[...],
                     w2[...], b2[...], a1r[...], a2r[...])
    h3, h4 = _encode(agg[1], agg[3], deg[...], w1[...], b1[...],
                     w2[...], b2[...], a1r[...], a2r[...])
    v = vb[...]
    p1 = jnp.dot(h1, v, preferred_element_type=jnp.float32)
    p2 = jnp.dot(h2, v, preferred_element_type=jnp.float32)
    p3 = jnp.dot(h3, v, preferred_element_type=jnp.float32)
    p4 = jnp.dot(h4, v, preferred_element_type=jnp.float32)
    out[...] = jnp.concatenate(
        [p2[:, 0:1], p1[:, 1:2], p4[:, 0:1], p3[:, 1:2]], axis=1) + bbr[...]


def _scores_call(agg, deg_col, w1, b1, w2, b2, a1r, a2r, vb, bb4):
    grid = N // _BLK
    blk = pl.BlockSpec((4, _BLK, 128), lambda i: (0, i, 0))
    full = pl.BlockSpec((128, 128), lambda i: (0, 0))
    row = pl.BlockSpec((1, 128), lambda i: (0, 0))
    one = pl.BlockSpec((1, 1), lambda i: (0, 0))
    return pl.pallas_call(
        _scores_body,
        grid=(grid,),
        in_specs=[blk, pl.BlockSpec((_BLK, 1), lambda i: (i, 0)),
                  full, row, full, row, one, one,
                  pl.BlockSpec((128, 2), lambda i: (0, 0)),
                  pl.BlockSpec((1, 4), lambda i: (0, 0))],
        out_specs=pl.BlockSpec((_BLK, 4), lambda i: (i, 0)),
        out_shape=jax.ShapeDtypeStruct((N, 4), jnp.float32),
    )(agg, deg_col, w1, b1, w2, b2, a1r, a2r, vb, bb4)


def kernel(edge_index, diff_edge_index, feat_a, feat_b, shuf_feat_a,
           shuf_feat_b, edge_weight, Wa, ba, Wb, bb, W1, b1, alpha1,
           W2, b2, alpha2, Wbil, bbil):
    src1, dst1 = edge_index[0], edge_index[1]
    src2, dst2 = diff_edge_index[0], diff_edge_index[1]
    pad = E2 - E
    pad_src = jnp.arange(pad, dtype=jnp.int32) % N
    pad_sink = jnp.full((pad,), NPAD - 1, jnp.int32)

    deg_idx = jnp.stack([
        jnp.concatenate([src1, pad_sink]),
        jnp.concatenate([dst1, pad_sink]),
    ]).reshape(NC, NS * KC, CH)
    deg = _deg_kernel(deg_idx)

    src1p = jnp.concatenate([src1, pad_src])
    src2p = jnp.concatenate([src2, pad_src])
    dst1p = jnp.concatenate([dst1, pad_sink])
    dst2p = jnp.concatenate([dst2, pad_sink])
    gidx = jnp.stack([src1p, src1p + N, src2p + 2 * N, src2p + 3 * N]
                     ).reshape(4, NS * KC, CH)
    sidx = jnp.stack([dst1p, dst2p]).reshape(2, NS * KC, CH)
    ewp = jnp.concatenate([edge_weight, jnp.zeros((pad,), jnp.float32)])

    tbl = _proj_call(feat_a, feat_b, shuf_feat_a, shuf_feat_b,
                     Wa, Wb, ba.reshape(1, 64), bb.reshape(1, 64),
                     deg[0, :N].reshape(N, 1)).reshape(4 * N, 128)

    agg = _agg_kernel(gidx, sidx, ewp, tbl)

    deg_in = deg[1].reshape(NPAD, 1)
    b1r = b1.reshape(1, 128)
    b2r = b2.reshape(1, 128)
    a1r = alpha1.reshape(1, 1)
    a2r = alpha2.reshape(1, 1)
    vb = _sums_call(agg, deg_in, W1, b1r, W2, b2r, a1r, a2r, Wbil)
    bb4 = jnp.broadcast_to(bbil.reshape(1, 1), (1, 4))
    scores = _scores_call(agg, deg_in, W1, b1r, W2, b2r, a1r, a2r, vb, bb4)
    return scores.T.reshape(-1)


# speedup vs baseline: 1.6902x; 1.0087x over previous
import functools

import jax
import jax.numpy as jnp
from jax import lax
from jax.experimental import pallas as pl
from jax.experimental.pallas import tpu as pltpu
from jax.experimental.pallas import tpu_sc as plsc

N = 10000
E = 320000
D = 128
NPAD = 10240
NC = 2
NS = 16
CH = 128
KC = 160
E2 = NS * KC * CH
RPT = NPAD // NS

_MESH = plsc.VectorSubcoreMesh(
    core_axis_name="c", subcore_axis_name="s", num_cores=NC, num_subcores=NS)


@functools.partial(
    pl.kernel,
    out_type=jax.ShapeDtypeStruct((NC, NPAD), jnp.float32),
    mesh=_MESH,
    scratch_types=[
        pltpu.VMEM((KC, CH), jnp.int32),
        pltpu.VMEM((CH,), jnp.float32),
        pltpu.VMEM((CH,), jnp.float32),
        pltpu.VMEM_SHARED((NPAD,), jnp.float32),
    ],
)
def _deg_kernel(idx_hbm, out_hbm, idx_v, ones_v, zeros_v, acc_sh):
    c = lax.axis_index("c")
    s = lax.axis_index("s")
    for j in range(CH // 16):
        ones_v[pl.ds(j * 16, 16)] = jnp.ones((16,), jnp.float32)
        zeros_v[pl.ds(j * 16, 16)] = jnp.zeros((16,), jnp.float32)
    for t in range(RPT // CH):
        pltpu.sync_copy(zeros_v, acc_sh.at[pl.ds(s * RPT + t * CH, CH)])
    plsc.subcore_barrier()
    pltpu.sync_copy(idx_hbm.at[c, pl.ds(s * KC, KC)], idx_v)

    def body(k, carry):
        pltpu.sync_copy(ones_v, acc_sh.at[idx_v.at[k]], add=True)
        return carry

    lax.fori_loop(0, KC, body, 0)
    plsc.subcore_barrier()
    for t in range(RPT // CH):
        r0 = s * RPT + t * CH
        pltpu.sync_copy(acc_sh.at[pl.ds(r0, CH)], out_hbm.at[c, pl.ds(r0, CH)])


SCH = 40
NSC = KC // SCH


@functools.partial(
    pl.kernel,
    out_type=jax.ShapeDtypeStruct((2 * NC, NPAD, D), jnp.float32),
    mesh=_MESH,
    scratch_types=[
        pltpu.VMEM((SCH, CH), jnp.int32),
        pltpu.VMEM((SCH, CH), jnp.int32),
        pltpu.VMEM((SCH * CH,), jnp.float32),
        pltpu.VMEM((CH, D), jnp.float32),
        pltpu.VMEM((CH, D), jnp.float32),
        pltpu.VMEM_SHARED((NPAD, D), jnp.float32),
        pltpu.SemaphoreType.DMA,
        pltpu.SemaphoreType.DMA,
        pltpu.SemaphoreType.DMA,
        pltpu.SemaphoreType.DMA,
    ],
)
def _agg_kernel(gidx_hbm, sidx_hbm, ew_hbm, tbl_hbm, out_hbm,
                gix_v, six_v, ew_v, buf0, buf1, acc_sh,
                sem0, sem1, ssem0, ssem1):
    c = lax.axis_index("c")
    s = lax.axis_index("s")
    bufs = (buf0, buf1)
    gsems = (sem0, sem1)
    ssems = (ssem0, ssem1)

    def zero_buf(i, carry):
        for j in range(D // 16):
            buf0[i, pl.ds(j * 16, 16)] = jnp.zeros((16,), jnp.float32)
        return carry

    for r in range(2):
        lax.fori_loop(0, CH, zero_buf, 0)
        for t in range(RPT // CH):
            pltpu.sync_copy(buf0, acc_sh.at[pl.ds(s * RPT + t * CH, CH)])
        plsc.subcore_barrier()

        plane = 2 * r + c

        def scale_rows(k, buf):
            def row16(ii, carry):
                wv = ew_v[pl.ds(k * CH + ii * 16, 16)]
                for l in range(16):
                    w = wv[l]
                    i = ii * 16 + l
                    for j in range(D // 16):
                        buf[i, pl.ds(j * 16, 16)] = (
                            buf[i, pl.ds(j * 16, 16)] * w)
                return carry
            lax.fori_loop(0, CH // 16, row16, 0)

        def superchunk(q, carry):
            base_row = s * KC + q * SCH
            pltpu.sync_copy(gidx_hbm.at[plane, pl.ds(base_row, SCH)], gix_v)
            pltpu.sync_copy(sidx_hbm.at[r, pl.ds(base_row, SCH)], six_v)
            if r == 1:
                pltpu.sync_copy(ew_hbm.at[pl.ds(base_row * CH, SCH * CH)],
                                ew_v)
            pltpu.async_copy(tbl_hbm.at[gix_v.at[0]], buf0, sem0)

            def pair(p, carry2):
                k0 = 2 * p
                pltpu.make_async_copy(
                    tbl_hbm.at[gix_v.at[k0]], buf0, gsems[0]).wait()

                @pl.when(k0 > 0)
                def _():
                    pltpu.make_async_copy(
                        buf1, acc_sh.at[six_v.at[0]], ssems[1]).wait()
                pltpu.async_copy(tbl_hbm.at[gix_v.at[k0 + 1]],
                                 buf1, gsems[1])
                if r == 1:
                    scale_rows(k0, buf0)
                pltpu.async_copy(buf0, acc_sh.at[six_v.at[k0]],
                                 ssems[0], add=True)
                pltpu.make_async_copy(
                    tbl_hbm.at[gix_v.at[k0 + 1]], buf1, gsems[1]).wait()
                pltpu.make_async_copy(
                    buf0, acc_sh.at[six_v.at[0]], ssems[0]).wait()

                @pl.when(k0 + 2 < SCH)
                def _():
                    pltpu.async_copy(tbl_hbm.at[gix_v.at[k0 + 2]],
                                     buf0, gsems[0])
                if r == 1:
                    scale_rows(k0 + 1, buf1)
                pltpu.async_copy(buf1, acc_sh.at[six_v.at[k0 + 1]],
                                 ssems[1], add=True)
                return carry2

            lax.fori_loop(0, SCH // 2, pair, 0)
            pltpu.make_async_copy(
                buf1, acc_sh.at[six_v.at[0]], ssems[1]).wait()
            return carry

        lax.fori_loop(0, NSC, superchunk, 0)

        plsc.subcore_barrier()
        for t in range(RPT // CH):
            r0 = s * RPT + t * CH
            pltpu.sync_copy(acc_sh.at[pl.ds(r0, CH)],
                            out_hbm.at[plane, pl.ds(r0, CH)])


def _proj_body(fa, fb, sa, sb, wa, wb, ba, bb, deg, out):
    pa = jnp.dot(fa[...], wa[...], preferred_element_type=jnp.float32) + ba[...]
    pb = jnp.dot(fb[...], wb[...], preferred_element_type=jnp.float32) + bb[...]
    proj = jnp.concatenate([pa, pb], axis=1)
    qa = jnp.dot(sa[...], wa[...], preferred_element_type=jnp.float32) + ba[...]
    qb = jnp.dot(sb[...], wb[...], preferred_element_type=jnp.float32) + bb[...]
    shuf = jnp.concatenate([qa, qb], axis=1)
    dg = deg[...]
    ns = jnp.where(dg > 0, lax.rsqrt(jnp.maximum(dg, 1.0)), 0.0)
    out[0] = proj * ns
    out[1] = shuf * ns
    out[2] = proj
    out[3] = shuf


_BLK = 1000


def _proj_call(fa, fb, sa, sb, wa, wb, ba, bb, deg_col):
    grid = N // _BLK
    row_spec = pl.BlockSpec((_BLK, 128), lambda i: (i, 0))
    full64 = pl.BlockSpec((128, 64), lambda i: (0, 0))
    bias = pl.BlockSpec((1, 64), lambda i: (0, 0))
    return pl.pallas_call(
        _proj_body,
        grid=(grid,),
        in_specs=[row_spec, row_spec, row_spec, row_spec,
                  full64, full64, bias, bias,
                  pl.BlockSpec((_BLK, 1), lambda i: (i, 0))],
        out_specs=pl.BlockSpec((4, _BLK, 128), lambda i: (0, i, 0)),
        out_shape=jax.ShapeDtypeStruct((4, N, 128), jnp.float32),
    )(fa, fb, sa, sb, wa, wb, ba, bb, deg_col)


def _encode(a0, a2, deg, w1, b1, w2, b2, a1r, a2r):
    nd = jnp.where(deg > 0, lax.rsqrt(jnp.maximum(deg, 1.0)), 0.0)
    z1 = jnp.dot(a0, w1, preferred_element_type=jnp.float32) * nd + b1
    z2 = jnp.dot(a2, w2, preferred_element_type=jnp.float32) + b2
    h1 = jnp.where(z1 > 0, z1, a1r * z1)
    h2 = jnp.where(z2 > 0, z2, a2r * z2)
    return h1, h2


def _post_body(agg, deg, w1, b1, w2, b2, a1r, a2r, wbil, bbr, out, acc, vbs):
    ph = pl.program_id(0)
    i = pl.program_id(1)

    @pl.when(ph == 0)
    def _():
        @pl.when(i == 0)
        def _():
            acc[...] = jnp.zeros_like(acc)

        h1, h2 = _encode(agg[0], agg[2], deg[...], w1[...], b1[...],
                         w2[...], b2[...], a1r[...], a2r[...])
        acc[0:1, :] += jnp.sum(h1, axis=0, keepdims=True)
        acc[1:2, :] += jnp.sum(h2, axis=0, keepdims=True)

        @pl.when(i == N // _BLK - 1)
        def _():
            c = jax.nn.sigmoid(acc[...] * (1.0 / N))
            eye = jnp.eye(128, dtype=jnp.float32)
            dnt = (((1,), (1,)), ((), ()))
            cc = lax.dot_general(eye, c, dnt,
                                 preferred_element_type=jnp.float32)
            vbs[...] = jnp.dot(wbil[...], cc,
                               preferred_element_type=jnp.float32)

    @pl.when(ph == 1)
    def _():
        h1, h2 = _encode(agg[0], agg[2], deg[...], w1[...], b1[...],
                         w2[...], b2[...], a1r[...], a2r[...])
        h3, h4 = _encode(agg[1], agg[3], deg[...], w1[...], b1[...],
                         w2[...], b2[...], a1r[...], a2r[...])
        v = vbs[...]
        p1 = jnp.dot(h1, v, preferred_element_type=jnp.float32)
        p2 = jnp.dot(h2, v, preferred_element_type=jnp.float32)
        p3 = jnp.dot(h3, v, preferred_element_type=jnp.float32)
        p4 = jnp.dot(h4, v, preferred_element_type=jnp.float32)
        out[...] = jnp.concatenate(
            [p2[:, 0:1], p1[:, 1:2], p4[:, 0:1], p3[:, 1:2]],
            axis=1) + bbr[...]


def _post_call(agg, deg_col, w1, b1, w2, b2, a1r, a2r, wbil, bb4):
    grid = N // _BLK
    blk = pl.BlockSpec((4, _BLK, 128), lambda p, i: (0, i, 0))
    full = pl.BlockSpec((128, 128), lambda p, i: (0, 0))
    row = pl.BlockSpec((1, 128), lambda p, i: (0, 0))
    one = pl.BlockSpec((1, 1), lambda p, i: (0, 0))
    return pl.pallas_call(
        _post_body,
        grid=(2, grid),
        in_specs=[blk, pl.BlockSpec((_BLK, 1), lambda p, i: (i, 0)),
                  full, row, full, row, one, one, full,
                  pl.BlockSpec((1, 4), lambda p, i: (0, 0))],
        out_specs=pl.BlockSpec((_BLK, 4), lambda p, i: (i, 0)),
        out_shape=jax.ShapeDtypeStruct((N, 4), jnp.float32),
        scratch_shapes=[pltpu.VMEM((2, 128), jnp.float32),
                        pltpu.VMEM((128, 2), jnp.float32)],
    )(agg, deg_col, w1, b1, w2, b2, a1r, a2r, wbil, bb4)


def kernel(edge_index, diff_edge_index, feat_a, feat_b, shuf_feat_a,
           shuf_feat_b, edge_weight, Wa, ba, Wb, bb, W1, b1, alpha1,
           W2, b2, alpha2, Wbil, bbil):
    src1, dst1 = edge_index[0], edge_index[1]
    src2, dst2 = diff_edge_index[0], diff_edge_index[1]
    pad = E2 - E
    pad_src = jnp.arange(pad, dtype=jnp.int32) % N
    pad_sink = jnp.full((pad,), NPAD - 1, jnp.int32)

    deg_idx = jnp.stack([
        jnp.concatenate([src1, pad_sink]),
        jnp.concatenate([dst1, pad_sink]),
    ]).reshape(NC, NS * KC, CH)
    deg = _deg_kernel(deg_idx)

    src1p = jnp.concatenate([src1, pad_src])
    src2p = jnp.concatenate([src2, pad_src])
    dst1p = jnp.concatenate([dst1, pad_sink])
    dst2p = jnp.concatenate([dst2, pad_sink])
    gidx = jnp.stack([src1p, src1p + N, src2p + 2 * N, src2p + 3 * N]
                     ).reshape(4, NS * KC, CH)
    sidx = jnp.stack([dst1p, dst2p]).reshape(2, NS * KC, CH)
    ewp = jnp.concatenate([edge_weight, jnp.zeros((pad,), jnp.float32)])

    tbl = _proj_call(feat_a, feat_b, shuf_feat_a, shuf_feat_b,
                     Wa, Wb, ba.reshape(1, 64), bb.reshape(1, 64),
                     deg[0, :N].reshape(N, 1)).reshape(4 * N, 128)

    agg = _agg_kernel(gidx, sidx, ewp, tbl)

    deg_in = deg[1].reshape(NPAD, 1)
    b1r = b1.reshape(1, 128)
    b2r = b2.reshape(1, 128)
    a1r = alpha1.reshape(1, 1)
    a2r = alpha2.reshape(1, 1)
    bb4 = jnp.broadcast_to(bbil.reshape(1, 1), (1, 4))
    scores = _post_call(agg, deg_in, W1, b1r, W2, b2r, a1r, a2r, Wbil, bb4)
    return scores.T.reshape(-1)
